# baseline (speedup 1.0000x reference)
import functools

import jax
import jax.numpy as jnp
from jax import lax
from jax.experimental import pallas as pl
from jax.experimental.pallas import tpu as pltpu

_EPS = 1e-5
_F32 = jnp.float32


def _densify_kernel(idx_ref, feats_ref, w_ref, grid_ref, occ_ref, y_ref):
    grid_ref[...] = jnp.zeros(grid_ref.shape, _F32)
    occ_ref[...] = jnp.zeros(occ_ref.shape, _F32)
    y_ref[...] = jnp.dot(feats_ref[...], w_ref[...],
                         preferred_element_type=_F32)
    n = idx_ref.shape[0]

    def body(i, carry):
        r = idx_ref[i]
        grid_ref[pl.ds(r, 1), :] = grid_ref[pl.ds(r, 1), :] + y_ref[pl.ds(i, 1), :]
        occ_ref[pl.ds(r, 1), :] = occ_ref[pl.ds(r, 1), :] + 1.0
        return carry

    lax.fori_loop(0, n, body, 0)


def _densify(idx, feats, w, rows):
    n, _ = feats.shape
    co = w.shape[1]
    return pl.pallas_call(
        _densify_kernel,
        out_shape=[
            jax.ShapeDtypeStruct((rows, co), _F32),
            jax.ShapeDtypeStruct((rows, 1), _F32),
        ],
        in_specs=[
            pl.BlockSpec(memory_space=pltpu.SMEM),
            pl.BlockSpec(),
            pl.BlockSpec(),
        ],
        out_specs=[pl.BlockSpec(), pl.BlockSpec()],
        scratch_shapes=[pltpu.VMEM((n, co), _F32)],
    )(idx, feats, w)


def _bn_kernel(grid_ref, occ_ref, g_ref, b_ref, out_ref, mask_ref):
    x = grid_ref[...]
    m = (occ_ref[...] > 0.0).astype(_F32)
    n = jnp.maximum(jnp.sum(m), 1.0)
    mean = jnp.sum(x * m, axis=0, keepdims=True) / n
    xc = (x - mean) * m
    var = jnp.sum(xc * xc, axis=0, keepdims=True) / n
    y = (x - mean) * lax.rsqrt(var + _EPS) * g_ref[...] + b_ref[...]
    out_ref[...] = y * m
    mask_ref[...] = m


def _bn_lateral(grid, occ, gamma, beta):
    rows, co = grid.shape
    return pl.pallas_call(
        _bn_kernel,
        out_shape=[
            jax.ShapeDtypeStruct((rows, co), _F32),
            jax.ShapeDtypeStruct((rows, 1), _F32),
        ],
    )(grid, occ, gamma.reshape(1, co), beta.reshape(1, co))


def _conv_kernel(P, W2, M, has_b, *refs):
    if has_b:
        xa_ref, xb_ref, w_ref, out_ref = refs
    else:
        xa_ref, w_ref, out_ref = refs
    for do in range(2):
        acc = jnp.zeros((P, 128), _F32)
        for di in range(2):
            bank = di - do + 1
            for kh in range(3):
                for kw in range(3):
                    off = M + (kh - 1) * W2 + (kw - 1)
                    xs = xa_ref[di, pl.ds(off, P), :]
                    if has_b:
                        xs = xs + xb_ref[di, pl.ds(off, P), :]
                    acc = acc + jnp.dot(xs, w_ref[bank, kh, kw],
                                        preferred_element_type=_F32)
        out_ref[do] = acc


def _conv(xa, xb, w, P, W2, M):
    has_b = xb is not None
    kern = functools.partial(_conv_kernel, P, W2, M, has_b)
    args = [xa] + ([xb] if has_b else []) + [w]
    return pl.pallas_call(
        kern,
        out_shape=jax.ShapeDtypeStruct((2, P, 128), _F32),
    )(*args)


def _mask_dilate_kernel(P, W2, M, has_b, *refs):
    if has_b:
        ma_ref, mb_ref, border_ref, mout_ref = refs
    else:
        ma_ref, border_ref, mout_ref = refs
    border = border_ref[...]
    for do in range(2):
        cnt = jnp.zeros((P, 1), _F32)
        for di in range(2):
            for kh in range(3):
                for kw in range(3):
                    off = M + (kh - 1) * W2 + (kw - 1)
                    ms = ma_ref[di, pl.ds(off, P), :]
                    if has_b:
                        ms = jnp.maximum(ms, mb_ref[di, pl.ds(off, P), :])
                    cnt = cnt + ms
        mout_ref[do] = (cnt > 0.0).astype(_F32) * border


def _mask_dilate(ma, mb, border, P, W2, M):
    has_b = mb is not None
    kern = functools.partial(_mask_dilate_kernel, P, W2, M, has_b)
    args = [ma] + ([mb] if has_b else []) + [border]
    return pl.pallas_call(
        kern,
        out_shape=jax.ShapeDtypeStruct((2, P, 1), _F32),
    )(*args)


def _bn_relu_kernel(has_b, *refs):
    if has_b:
        x_ref, ma_ref, mb_ref, g_ref, b_ref, out_ref = refs
    else:
        x_ref, ma_ref, g_ref, b_ref, out_ref = refs
    n = 0.0
    s1 = jnp.zeros((1, 128), _F32)
    for do in range(2):
        m = ma_ref[do]
        if has_b:
            m = jnp.maximum(m, mb_ref[do])
        n = n + jnp.sum(m)
        s1 = s1 + jnp.sum(x_ref[do] * m, axis=0, keepdims=True)
    n = jnp.maximum(n, 1.0)
    mean = s1 / n
    v = jnp.zeros((1, 128), _F32)
    for do in range(2):
        m = ma_ref[do]
        if has_b:
            m = jnp.maximum(m, mb_ref[do])
        xc = (x_ref[do] - mean) * m
        v = v + jnp.sum(xc * xc, axis=0, keepdims=True)
    rs = lax.rsqrt(v / n + _EPS)
    g = g_ref[...]
    b = b_ref[...]
    for do in range(2):
        m = ma_ref[do]
        if has_b:
            m = jnp.maximum(m, mb_ref[do])
        y = (x_ref[do] - mean) * rs * g + b
        out_ref[do] = jnp.maximum(y, 0.0) * m


def _bn_relu(x, ma, mb, gamma, beta):
    has_b = mb is not None
    P = x.shape[1]
    kern = functools.partial(_bn_relu_kernel, has_b)
    args = [x, ma] + ([mb] if has_b else [])
    args += [gamma.reshape(1, 128), beta.reshape(1, 128)]
    return pl.pallas_call(
        kern,
        out_shape=jax.ShapeDtypeStruct((2, P, 128), _F32),
    )(*args)


def _flat_padded_idx(coords, H, W):
    w2 = W + 2
    plane = (H + 2) * w2
    d = coords[:, 1]
    h = coords[:, 2]
    w = coords[:, 3]
    return (d * plane + (h + 1) * w2 + (w + 1)).astype(jnp.int32)


def _dilate2(x):
    d, h, w, c = x.shape
    z = jnp.zeros_like(x)
    xw = jnp.stack([x, z], axis=3).reshape(d, h, 2 * w, c)[:, :, :2 * w - 1, :]
    zh = jnp.zeros_like(xw)
    xh = jnp.stack([xw, zh], axis=2).reshape(d, 2 * h, 2 * w - 1, c)[:, :2 * h - 1, :, :]
    return xh


def _embed(x, margin):
    d, h, w, c = x.shape
    xp = jnp.pad(x, ((0, 0), (1, 1), (1, 1), (0, 0)))
    xp = xp.reshape(d, (h + 2) * (w + 2), c)
    return jnp.pad(xp, ((0, 0), (margin, margin), (0, 0)))


def _margin(xflat, P, margin):
    c = xflat.shape[1]
    return jnp.pad(xflat.reshape(2, P, c), ((0, 0), (margin, margin), (0, 0)))


def _inner(xflat, H, W):
    c = xflat.shape[1]
    x = xflat.reshape(2, H + 2, W + 2, c)
    return x[:, 1:H + 1, 1:W + 1, :]


def _border_mask(H, W):
    hh = jnp.arange(H + 2)
    ww = jnp.arange(W + 2)
    bh = (hh >= 1) & (hh <= H)
    bw = (ww >= 1) & (ww <= W)
    b = (bh[:, None] & bw[None, :]).astype(_F32)
    return b.reshape((H + 2) * (W + 2), 1)


def kernel(c2_feats, c3_feats, c4_feats, lat2_w, lat2_g, lat2_b, lat3_w,
           lat3_g, lat3_b, lat4_w, lat4_g, lat4_b, up43_w, up43_g, up43_b,
           up32_w, up32_g, up32_b, out_w, out_g, out_b, c2_coords, c3_coords,
           c4_coords):
    H4 = W4 = 24
    H3 = W3 = 47
    H2 = W2_ = 93
    P4 = (H4 + 2) * (W4 + 2)
    P3 = (H3 + 2) * (W3 + 2)
    P2 = (H2 + 2) * (W2_ + 2)
    W2p3 = W3 + 2
    W2p2 = W2_ + 2
    M3 = 56
    M2 = 96

    idx2 = _flat_padded_idx(c2_coords, H2, W2_)
    idx3 = _flat_padded_idx(c3_coords, H3, W3)
    idx4 = _flat_padded_idx(c4_coords, H4, W4)
    grid2, occ2 = _densify(idx2, c2_feats, lat2_w, 2 * P2)
    grid3, occ3 = _densify(idx3, c3_feats, lat3_w, 2 * P3)
    grid4, occ4 = _densify(idx4, c4_feats, lat4_w, 2 * P4)

    p2_lat, c2m = _bn_lateral(grid2, occ2, lat2_g, lat2_b)
    p3_lat, c3m = _bn_lateral(grid3, occ3, lat3_g, lat3_b)
    p4, c4m = _bn_lateral(grid4, occ4, lat4_g, lat4_b)

    border3 = _border_mask(H3, W3)
    border2 = _border_mask(H2, W2_)

    xa = _embed(_dilate2(_inner(p4, H4, W4)), M3)
    ma = _embed(_dilate2(_inner(c4m, H4, W4)), M3)
    raw = _conv(xa, None, up43_w, P3, W2p3, M3)
    p4_up_m = _mask_dilate(ma, None, border3, P3, W2p3, M3)
    p4_up = _bn_relu(raw, p4_up_m, None, up43_g, up43_b)
    p4_up = p4_up.reshape(2 * P3, 128)
    p4_up_m = p4_up_m.reshape(2 * P3, 1)

    xa = _embed(_dilate2(_inner(p3_lat, H3, W3)), M2)
    xb = _embed(_dilate2(_inner(p4_up, H3, W3)), M2)
    ma = _embed(_dilate2(_inner(c3m, H3, W3)), M2)
    mb = _embed(_dilate2(_inner(p4_up_m, H3, W3)), M2)
    raw = _conv(xa, xb, up32_w, P2, W2p2, M2)
    p3_up_m = _mask_dilate(ma, mb, border2, P2, W2p2, M2)
    p3_up = _bn_relu(raw, p3_up_m, None, up32_g, up32_b)
    p3_up = p3_up.reshape(2 * P2, 128)
    p3_up_m = p3_up_m.reshape(2 * P2, 1)

    xa = _margin(p2_lat, P2, M2)
    xb = _margin(p3_up, P2, M2)
    raw = _conv(xa, xb, out_w, P2, W2p2, M2)
    y = _bn_relu(raw, c2m.reshape(2, P2, 1), p3_up_m.reshape(2, P2, 1),
                 out_g, out_b)

    out = y.reshape(2, H2 + 2, W2_ + 2, 128)[:, 1:H2 + 1, 1:W2_ + 1, :]
    return out[None]


# speedup vs baseline: 1.0111x; 1.0111x over previous
import functools

import jax
import jax.numpy as jnp
from jax import lax
from jax.experimental import pallas as pl
from jax.experimental.pallas import tpu as pltpu
from jax.experimental.pallas import tpu_sc as plsc

_EPS = 1e-5
_F32 = jnp.float32


def _mm_kernel(x_ref, w_ref, o_ref):
    o_ref[...] = jnp.dot(x_ref[...], w_ref[...], preferred_element_type=_F32)


def _mm(x, w):
    return pl.pallas_call(
        _mm_kernel,
        out_shape=jax.ShapeDtypeStruct((x.shape[0], w.shape[1]), _F32),
    )(x, w)


def _sc_scatter_all(levels):
    max_rh = max(rh for (_, _, _, rh) in levels)
    max_ck = max(y.shape[0] // 16 for (y, _, _, _) in levels)
    max_k = max_ck // 128
    mesh = plsc.VectorSubcoreMesh(core_axis_name="c", subcore_axis_name="s")
    nlev = len(levels)

    def body(*refs):
        y_hbms = refs[0:nlev]
        idx_hbms = refs[nlev:2 * nlev]
        zg_hbm = refs[2 * nlev]
        ones_hbm = refs[2 * nlev + 1]
        outs = refs[2 * nlev + 2:4 * nlev + 2]
        idx_v, idx2_v, y_v, ones_v, grid_sh = refs[4 * nlev + 2:]
        c = lax.axis_index("c")
        s = lax.axis_index("s")
        pltpu.sync_copy(ones_hbm, ones_v)
        for li, (y_arr, _, plane, rhpad) in enumerate(levels):
            npad = y_arr.shape[0]
            c_chunk = npad // 16
            k = c_chunk // 128
            rpt = rhpad // 16
            pltpu.sync_copy(zg_hbm.at[pl.ds(s * rpt, rpt)],
                            grid_sh.at[pl.ds(s * rpt, rpt)])
            pltpu.sync_copy(idx_hbms[li].at[pl.ds(s * c_chunk, c_chunk)],
                            idx_v.at[pl.ds(0, c_chunk)])
            base = c * plane
            for j in range(k):
                for l in range(8):
                    v = idx_v[pl.ds(j * 128 + l * 16, 16)]
                    local = v - base
                    inb = (local >= 0) & (local < plane)
                    idx2_v[j, pl.ds(l * 16, 16)] = jnp.where(
                        inb, local, rhpad - 1)
            plsc.subcore_barrier()
            for j in range(k):
                pltpu.sync_copy(
                    y_hbms[li].at[pl.ds(s * c_chunk + j * 128, 128)], y_v)
                pltpu.sync_copy(y_v, grid_sh.at[idx2_v.at[j]], add=True)
            plsc.subcore_barrier()
            pltpu.sync_copy(grid_sh.at[pl.ds(s * rpt, rpt)],
                            outs[2 * li].at[pl.ds(c * rhpad + s * rpt, rpt)])
            pltpu.sync_copy(zg_hbm.at[pl.ds(s * rpt, rpt)],
                            grid_sh.at[pl.ds(s * rpt, rpt)])
            plsc.subcore_barrier()
            for j in range(k):
                pltpu.sync_copy(ones_v, grid_sh.at[idx2_v.at[j]], add=True)
            plsc.subcore_barrier()
            pltpu.sync_copy(
                grid_sh.at[pl.ds(s * rpt, rpt)],
                outs[2 * li + 1].at[pl.ds(c * rhpad + s * rpt, rpt)])
            plsc.subcore_barrier()

    zg = jnp.zeros((max_rh, 128), _F32)
    ones = jnp.ones((128, 128), _F32)
    out_types = []
    for (_, _, _, rh) in levels:
        out_types.append(jax.ShapeDtypeStruct((2 * rh, 128), _F32))
        out_types.append(jax.ShapeDtypeStruct((2 * rh, 128), _F32))
    call = pl.kernel(
        body,
        mesh=mesh,
        out_type=out_types,
        scratch_types=[
            pltpu.VMEM((max_ck,), jnp.int32),
            pltpu.VMEM((max_k, 128), jnp.int32),
            pltpu.VMEM((128, 128), _F32),
            pltpu.VMEM((128, 128), _F32),
            pltpu.VMEM_SHARED((max_rh, 128), _F32),
        ],
    )
    args = ([y for (y, _, _, _) in levels]
            + [ix for (_, ix, _, _) in levels] + [zg, ones])
    return call(*args)


def _densify_prep(idx, feats, w):
    n = feats.shape[0]
    npad = -(-n // 2048) * 2048
    y = _mm(feats, w)
    y_pad = jnp.pad(y, ((0, npad - n), (0, 0)))
    idx_pad = jnp.pad(idx, (0, npad - n), constant_values=(1 << 28))
    return y_pad, idx_pad


def _densify_split(raw, plane, rhpad):
    return jnp.concatenate([raw[:plane], raw[rhpad:rhpad + plane]], axis=0)


def _bn_kernel(grid_ref, occ_ref, g_ref, b_ref, out_ref, mask_ref):
    x = grid_ref[...]
    m = (occ_ref[...][:, 0:1] > 0.0).astype(_F32)
    n = jnp.maximum(jnp.sum(m), 1.0)
    mean = jnp.sum(x * m, axis=0, keepdims=True) / n
    xc = (x - mean) * m
    var = jnp.sum(xc * xc, axis=0, keepdims=True) / n
    y = (x - mean) * lax.rsqrt(var + _EPS) * g_ref[...] + b_ref[...]
    out_ref[...] = y * m
    mask_ref[...] = m


def _bn_lateral(grid, occ, gamma, beta):
    rows, co = grid.shape
    return pl.pallas_call(
        _bn_kernel,
        out_shape=[
            jax.ShapeDtypeStruct((rows, co), _F32),
            jax.ShapeDtypeStruct((rows, 1), _F32),
        ],
    )(grid, occ, gamma.reshape(1, co), beta.reshape(1, co))


def _conv_kernel(P, W2, M, has_b, *refs):
    if has_b:
        xa_ref, xb_ref, w_ref, out_ref = refs
    else:
        xa_ref, w_ref, out_ref = refs
    for do in range(2):
        acc = jnp.zeros((P, 128), _F32)
        for di in range(2):
            bank = di - do + 1
            for kh in range(3):
                for kw in range(3):
                    off = M + (kh - 1) * W2 + (kw - 1)
                    xs = xa_ref[di, pl.ds(off, P), :]
                    if has_b:
                        xs = xs + xb_ref[di, pl.ds(off, P), :]
                    acc = acc + jnp.dot(xs, w_ref[bank, kh, kw],
                                        preferred_element_type=_F32)
        out_ref[do] = acc


def _conv(xa, xb, w, P, W2, M):
    has_b = xb is not None
    kern = functools.partial(_conv_kernel, P, W2, M, has_b)
    args = [xa] + ([xb] if has_b else []) + [w]
    return pl.pallas_call(
        kern,
        out_shape=jax.ShapeDtypeStruct((2, P, 128), _F32),
    )(*args)


def _mask_dilate_kernel(P, W2, M, has_b, *refs):
    if has_b:
        ma_ref, mb_ref, border_ref, mout_ref = refs
    else:
        ma_ref, border_ref, mout_ref = refs
    border = border_ref[...]
    for do in range(2):
        cnt = jnp.zeros((P, 1), _F32)
        for di in range(2):
            for kh in range(3):
                for kw in range(3):
                    off = M + (kh - 1) * W2 + (kw - 1)
                    ms = ma_ref[di, pl.ds(off, P), :]
                    if has_b:
                        ms = jnp.maximum(ms, mb_ref[di, pl.ds(off, P), :])
                    cnt = cnt + ms
        mout_ref[do] = (cnt > 0.0).astype(_F32) * border


def _mask_dilate(ma, mb, border, P, W2, M):
    has_b = mb is not None
    kern = functools.partial(_mask_dilate_kernel, P, W2, M, has_b)
    args = [ma] + ([mb] if has_b else []) + [border]
    return pl.pallas_call(
        kern,
        out_shape=jax.ShapeDtypeStruct((2, P, 1), _F32),
    )(*args)


def _bn_relu_kernel(has_b, *refs):
    if has_b:
        x_ref, ma_ref, mb_ref, g_ref, b_ref, out_ref = refs
    else:
        x_ref, ma_ref, g_ref, b_ref, out_ref = refs
    n = 0.0
    s1 = jnp.zeros((1, 128), _F32)
    for do in range(2):
        m = ma_ref[do]
        if has_b:
            m = jnp.maximum(m, mb_ref[do])
        n = n + jnp.sum(m)
        s1 = s1 + jnp.sum(x_ref[do] * m, axis=0, keepdims=True)
    n = jnp.maximum(n, 1.0)
    mean = s1 / n
    v = jnp.zeros((1, 128), _F32)
    for do in range(2):
        m = ma_ref[do]
        if has_b:
            m = jnp.maximum(m, mb_ref[do])
        xc = (x_ref[do] - mean) * m
        v = v + jnp.sum(xc * xc, axis=0, keepdims=True)
    rs = lax.rsqrt(v / n + _EPS)
    g = g_ref[...]
    b = b_ref[...]
    for do in range(2):
        m = ma_ref[do]
        if has_b:
            m = jnp.maximum(m, mb_ref[do])
        y = (x_ref[do] - mean) * rs * g + b
        out_ref[do] = jnp.maximum(y, 0.0) * m


def _bn_relu(x, ma, mb, gamma, beta):
    has_b = mb is not None
    P = x.shape[1]
    kern = functools.partial(_bn_relu_kernel, has_b)
    args = [x, ma] + ([mb] if has_b else [])
    args += [gamma.reshape(1, 128), beta.reshape(1, 128)]
    return pl.pallas_call(
        kern,
        out_shape=jax.ShapeDtypeStruct((2, P, 128), _F32),
    )(*args)


def _flat_padded_idx(coords, H, W):
    w2 = W + 2
    plane = (H + 2) * w2
    d = coords[:, 1]
    h = coords[:, 2]
    w = coords[:, 3]
    return (d * plane + (h + 1) * w2 + (w + 1)).astype(jnp.int32)


def _dilate2(x):
    d, h, w, c = x.shape
    z = jnp.zeros_like(x)
    xw = jnp.stack([x, z], axis=3).reshape(d, h, 2 * w, c)[:, :, :2 * w - 1, :]
    zh = jnp.zeros_like(xw)
    xh = jnp.stack([xw, zh], axis=2).reshape(d, 2 * h, 2 * w - 1, c)[:, :2 * h - 1, :, :]
    return xh


def _embed(x, margin):
    d, h, w, c = x.shape
    xp = jnp.pad(x, ((0, 0), (1, 1), (1, 1), (0, 0)))
    xp = xp.reshape(d, (h + 2) * (w + 2), c)
    return jnp.pad(xp, ((0, 0), (margin, margin), (0, 0)))


def _margin(xflat, P, margin):
    c = xflat.shape[1]
    return jnp.pad(xflat.reshape(2, P, c), ((0, 0), (margin, margin), (0, 0)))


def _inner(xflat, H, W):
    c = xflat.shape[1]
    x = xflat.reshape(2, H + 2, W + 2, c)
    return x[:, 1:H + 1, 1:W + 1, :]


def _border_mask(H, W):
    hh = jnp.arange(H + 2)
    ww = jnp.arange(W + 2)
    bh = (hh >= 1) & (hh <= H)
    bw = (ww >= 1) & (ww <= W)
    b = (bh[:, None] & bw[None, :]).astype(_F32)
    return b.reshape((H + 2) * (W + 2), 1)


def kernel(c2_feats, c3_feats, c4_feats, lat2_w, lat2_g, lat2_b, lat3_w,
           lat3_g, lat3_b, lat4_w, lat4_g, lat4_b, up43_w, up43_g, up43_b,
           up32_w, up32_g, up32_b, out_w, out_g, out_b, c2_coords, c3_coords,
           c4_coords):
    H4 = W4 = 24
    H3 = W3 = 47
    H2 = W2_ = 93
    P4 = (H4 + 2) * (W4 + 2)
    P3 = (H3 + 2) * (W3 + 2)
    P2 = (H2 + 2) * (W2_ + 2)
    W2p3 = W3 + 2
    W2p2 = W2_ + 2
    M3 = 56
    M2 = 96

    idx2 = _flat_padded_idx(c2_coords, H2, W2_)
    idx3 = _flat_padded_idx(c3_coords, H3, W3)
    idx4 = _flat_padded_idx(c4_coords, H4, W4)
    R2 = -(-P2 // 128) * 128
    R3 = -(-P3 // 128) * 128
    R4 = -(-P4 // 128) * 128
    y2, ix2 = _densify_prep(idx2, c2_feats, lat2_w)
    y3, ix3 = _densify_prep(idx3, c3_feats, lat3_w)
    y4, ix4 = _densify_prep(idx4, c4_feats, lat4_w)
    g2, c2c, g3, c3c, g4, c4c = _sc_scatter_all([
        (y2, ix2, P2, R2), (y3, ix3, P3, R3), (y4, ix4, P4, R4)])
    grid2, occ2 = _densify_split(g2, P2, R2), _densify_split(c2c, P2, R2)
    grid3, occ3 = _densify_split(g3, P3, R3), _densify_split(c3c, P3, R3)
    grid4, occ4 = _densify_split(g4, P4, R4), _densify_split(c4c, P4, R4)

    p2_lat, c2m = _bn_lateral(grid2, occ2, lat2_g, lat2_b)
    p3_lat, c3m = _bn_lateral(grid3, occ3, lat3_g, lat3_b)
    p4, c4m = _bn_lateral(grid4, occ4, lat4_g, lat4_b)

    border3 = _border_mask(H3, W3)
    border2 = _border_mask(H2, W2_)

    xa = _embed(_dilate2(_inner(p4, H4, W4)), M3)
    ma = _embed(_dilate2(_inner(c4m, H4, W4)), M3)
    raw = _conv(xa, None, up43_w, P3, W2p3, M3)
    p4_up_m = _mask_dilate(ma, None, border3, P3, W2p3, M3)
    p4_up = _bn_relu(raw, p4_up_m, None, up43_g, up43_b)
    p4_up = p4_up.reshape(2 * P3, 128)
    p4_up_m = p4_up_m.reshape(2 * P3, 1)

    xa = _embed(_dilate2(_inner(p3_lat, H3, W3)), M2)
    xb = _embed(_dilate2(_inner(p4_up, H3, W3)), M2)
    ma = _embed(_dilate2(_inner(c3m, H3, W3)), M2)
    mb = _embed(_dilate2(_inner(p4_up_m, H3, W3)), M2)
    raw = _conv(xa, xb, up32_w, P2, W2p2, M2)
    p3_up_m = _mask_dilate(ma, mb, border2, P2, W2p2, M2)
    p3_up = _bn_relu(raw, p3_up_m, None, up32_g, up32_b)
    p3_up = p3_up.reshape(2 * P2, 128)
    p3_up_m = p3_up_m.reshape(2 * P2, 1)

    xa = _margin(p2_lat, P2, M2)
    xb = _margin(p3_up, P2, M2)
    raw = _conv(xa, xb, out_w, P2, W2p2, M2)
    y = _bn_relu(raw, c2m.reshape(2, P2, 1), p3_up_m.reshape(2, P2, 1),
                 out_g, out_b)

    out = y.reshape(2, H2 + 2, W2_ + 2, 128)[:, 1:H2 + 1, 1:W2_ + 1, :]
    return out[None]


# speedup vs baseline: 1.0922x; 1.0802x over previous
import functools

import jax
import jax.numpy as jnp
from jax import lax
from jax.experimental import pallas as pl
from jax.experimental.pallas import tpu as pltpu
from jax.experimental.pallas import tpu_sc as plsc

_EPS = 1e-5
_F32 = jnp.float32


def _mm_kernel(x_ref, w_ref, o_ref):
    o_ref[...] = jnp.dot(x_ref[...], w_ref[...], preferred_element_type=_F32)


def _mm(x, w):
    return pl.pallas_call(
        _mm_kernel,
        out_shape=jax.ShapeDtypeStruct((x.shape[0], w.shape[1]), _F32),
    )(x, w)


def _sc_scatter_all(levels):
    max_rh = max(rh for (_, _, _, rh) in levels)
    max_ck = max(y.shape[0] // 16 for (y, _, _, _) in levels)
    max_k = max_ck // 128
    mesh = plsc.VectorSubcoreMesh(core_axis_name="c", subcore_axis_name="s")
    nlev = len(levels)

    def body(*refs):
        y_hbms = refs[0:nlev]
        idx_hbms = refs[nlev:2 * nlev]
        zg_hbm = refs[2 * nlev]
        ones_hbm = refs[2 * nlev + 1]
        outs = refs[2 * nlev + 2:4 * nlev + 2]
        idx_v, idx2_v, y_v, ones_v, grid_sh = refs[4 * nlev + 2:]
        c = lax.axis_index("c")
        s = lax.axis_index("s")
        pltpu.sync_copy(ones_hbm, ones_v)
        for li, (y_arr, _, plane, rhpad) in enumerate(levels):
            npad = y_arr.shape[0]
            c_chunk = npad // 16
            k = c_chunk // 128
            rpt = rhpad // 16
            pltpu.sync_copy(zg_hbm.at[pl.ds(s * rpt, rpt)],
                            grid_sh.at[pl.ds(s * rpt, rpt)])
            pltpu.sync_copy(idx_hbms[li].at[pl.ds(s * c_chunk, c_chunk)],
                            idx_v.at[pl.ds(0, c_chunk)])
            base = c * plane
            for j in range(k):
                for l in range(8):
                    v = idx_v[pl.ds(j * 128 + l * 16, 16)]
                    local = v - base
                    inb = (local >= 0) & (local < plane)
                    idx2_v[j, pl.ds(l * 16, 16)] = jnp.where(
                        inb, local, rhpad - 1)
            plsc.subcore_barrier()
            for j in range(k):
                pltpu.sync_copy(
                    y_hbms[li].at[pl.ds(s * c_chunk + j * 128, 128)], y_v)
                pltpu.sync_copy(y_v, grid_sh.at[idx2_v.at[j]], add=True)
            plsc.subcore_barrier()
            pltpu.sync_copy(grid_sh.at[pl.ds(s * rpt, rpt)],
                            outs[2 * li].at[pl.ds(c * rhpad + s * rpt, rpt)])
            pltpu.sync_copy(zg_hbm.at[pl.ds(s * rpt, rpt)],
                            grid_sh.at[pl.ds(s * rpt, rpt)])
            plsc.subcore_barrier()
            for j in range(k):
                pltpu.sync_copy(ones_v, grid_sh.at[idx2_v.at[j]], add=True)
            plsc.subcore_barrier()
            pltpu.sync_copy(
                grid_sh.at[pl.ds(s * rpt, rpt)],
                outs[2 * li + 1].at[pl.ds(c * rhpad + s * rpt, rpt)])
            plsc.subcore_barrier()

    zg = jnp.zeros((max_rh, 128), _F32)
    ones = jnp.ones((128, 128), _F32)
    out_types = []
    for (_, _, _, rh) in levels:
        out_types.append(jax.ShapeDtypeStruct((2 * rh, 128), _F32))
        out_types.append(jax.ShapeDtypeStruct((2 * rh, 128), _F32))
    call = pl.kernel(
        body,
        mesh=mesh,
        out_type=out_types,
        scratch_types=[
            pltpu.VMEM((max_ck,), jnp.int32),
            pltpu.VMEM((max_k, 128), jnp.int32),
            pltpu.VMEM((128, 128), _F32),
            pltpu.VMEM((128, 128), _F32),
            pltpu.VMEM_SHARED((max_rh, 128), _F32),
        ],
    )
    args = ([y for (y, _, _, _) in levels]
            + [ix for (_, ix, _, _) in levels] + [zg, ones])
    return call(*args)


def _densify_prep(idx, feats, w):
    n = feats.shape[0]
    npad = -(-n // 2048) * 2048
    y = _mm(feats, w)
    y_pad = jnp.pad(y, ((0, npad - n), (0, 0)))
    idx_pad = jnp.pad(idx, (0, npad - n), constant_values=(1 << 28))
    return y_pad, idx_pad


def _densify_split(raw, plane, rhpad):
    return jnp.concatenate([raw[:plane], raw[rhpad:rhpad + plane]], axis=0)


def _bn_kernel(margin, grid_ref, occ_ref, g_ref, b_ref, out_ref, mask_ref):
    x = grid_ref[...]
    m = (occ_ref[...][:, 0:1] > 0.0).astype(_F32)
    n = jnp.maximum(jnp.sum(m), 1.0)
    mean = jnp.sum(x * m, axis=0, keepdims=True) / n
    xc = (x - mean) * m
    var = jnp.sum(xc * xc, axis=0, keepdims=True) / n
    y = (x - mean) * lax.rsqrt(var + _EPS) * g_ref[...] + b_ref[...]
    y = y * m
    if margin is None:
        out_ref[...] = y
    else:
        p = grid_ref.shape[0] // 2
        z = jnp.zeros((margin, 128), _F32)
        for do in range(2):
            out_ref[do, pl.ds(0, margin), :] = z
            out_ref[do, pl.ds(margin, p), :] = y[do * p:(do + 1) * p]
            out_ref[do, pl.ds(margin + p, margin), :] = z
    mask_ref[...] = m


def _bn_lateral(grid, occ, gamma, beta, margin=None):
    rows, co = grid.shape
    if margin is None:
        out_sh = jax.ShapeDtypeStruct((rows, co), _F32)
    else:
        out_sh = jax.ShapeDtypeStruct((2, rows // 2 + 2 * margin, co), _F32)
    return pl.pallas_call(
        functools.partial(_bn_kernel, margin),
        out_shape=[out_sh, jax.ShapeDtypeStruct((rows, 1), _F32)],
    )(grid, occ, gamma.reshape(1, co), beta.reshape(1, co))


def _conv_kernel(P, W2, M, has_b, *refs):
    if has_b:
        xa_ref, xb_ref, w_ref, out_ref = refs
    else:
        xa_ref, w_ref, out_ref = refs
    for do in range(2):
        acc = jnp.zeros((P, 128), _F32)
        for di in range(2):
            bank = di - do + 1
            for kh in range(3):
                for kw in range(3):
                    off = M + (kh - 1) * W2 + (kw - 1)
                    xs = xa_ref[di, pl.ds(off, P), :]
                    if has_b:
                        xs = xs + xb_ref[di, pl.ds(off, P), :]
                    acc = acc + jnp.dot(xs, w_ref[bank, kh, kw],
                                        preferred_element_type=_F32)
        out_ref[do] = acc


def _conv(xa, xb, w, P, W2, M):
    has_b = xb is not None
    kern = functools.partial(_conv_kernel, P, W2, M, has_b)
    args = [xa] + ([xb] if has_b else []) + [w]
    return pl.pallas_call(
        kern,
        out_shape=jax.ShapeDtypeStruct((2, P, 128), _F32),
    )(*args)


def _mask_dilate_kernel(P, W2, M, has_b, *refs):
    if has_b:
        ma_ref, mb_ref, border_ref, mout_ref = refs
    else:
        ma_ref, border_ref, mout_ref = refs
    border = border_ref[...]
    for do in range(2):
        cnt = jnp.zeros((P, 1), _F32)
        for di in range(2):
            for kh in range(3):
                for kw in range(3):
                    off = M + (kh - 1) * W2 + (kw - 1)
                    ms = ma_ref[di, pl.ds(off, P), :]
                    if has_b:
                        ms = jnp.maximum(ms, mb_ref[di, pl.ds(off, P), :])
                    cnt = cnt + ms
        mout_ref[do] = (cnt > 0.0).astype(_F32) * border


def _mask_dilate(ma, mb, border, P, W2, M):
    has_b = mb is not None
    kern = functools.partial(_mask_dilate_kernel, P, W2, M, has_b)
    args = [ma] + ([mb] if has_b else []) + [border]
    return pl.pallas_call(
        kern,
        out_shape=jax.ShapeDtypeStruct((2, P, 1), _F32),
    )(*args)


def _bn_relu_kernel(has_b, margin, *refs):
    if has_b:
        x_ref, ma_ref, mb_ref, g_ref, b_ref, out_ref = refs
    else:
        x_ref, ma_ref, g_ref, b_ref, out_ref = refs
    n = 0.0
    s1 = jnp.zeros((1, 128), _F32)
    for do in range(2):
        m = ma_ref[do]
        if has_b:
            m = jnp.maximum(m, mb_ref[do])
        n = n + jnp.sum(m)
        s1 = s1 + jnp.sum(x_ref[do] * m, axis=0, keepdims=True)
    n = jnp.maximum(n, 1.0)
    mean = s1 / n
    v = jnp.zeros((1, 128), _F32)
    for do in range(2):
        m = ma_ref[do]
        if has_b:
            m = jnp.maximum(m, mb_ref[do])
        xc = (x_ref[do] - mean) * m
        v = v + jnp.sum(xc * xc, axis=0, keepdims=True)
    rs = lax.rsqrt(v / n + _EPS)
    g = g_ref[...]
    b = b_ref[...]
    p = x_ref.shape[1]
    for do in range(2):
        m = ma_ref[do]
        if has_b:
            m = jnp.maximum(m, mb_ref[do])
        y = (x_ref[do] - mean) * rs * g + b
        y = jnp.maximum(y, 0.0) * m
        if margin is None:
            out_ref[do] = y
        else:
            z = jnp.zeros((margin, 128), _F32)
            out_ref[do, pl.ds(0, margin), :] = z
            out_ref[do, pl.ds(margin, p), :] = y
            out_ref[do, pl.ds(margin + p, margin), :] = z


def _bn_relu(x, ma, mb, gamma, beta, margin=None):
    has_b = mb is not None
    P = x.shape[1]
    kern = functools.partial(_bn_relu_kernel, has_b, margin)
    args = [x, ma] + ([mb] if has_b else [])
    args += [gamma.reshape(1, 128), beta.reshape(1, 128)]
    rows = P if margin is None else P + 2 * margin
    return pl.pallas_call(
        kern,
        out_shape=jax.ShapeDtypeStruct((2, rows, 128), _F32),
    )(*args)


def _flat_padded_idx(coords, H, W):
    w2 = W + 2
    plane = (H + 2) * w2
    d = coords[:, 1]
    h = coords[:, 2]
    w = coords[:, 3]
    return (d * plane + (h + 1) * w2 + (w + 1)).astype(jnp.int32)


def _dilate2(x):
    d, h, w, c = x.shape
    z = jnp.zeros_like(x)
    xw = jnp.stack([x, z], axis=3).reshape(d, h, 2 * w, c)[:, :, :2 * w - 1, :]
    zh = jnp.zeros_like(xw)
    xh = jnp.stack([xw, zh], axis=2).reshape(d, 2 * h, 2 * w - 1, c)[:, :2 * h - 1, :, :]
    return xh


def _embed(x):
    d, h, w, c = x.shape
    xp = jnp.pad(x, ((0, 0), (3, 3), (1, 1), (0, 0)))
    return xp.reshape(d, (h + 6) * (w + 2), c)


def _inner(xflat, H, W):
    c = xflat.shape[1]
    x = xflat.reshape(2, H + 2, W + 2, c)
    return x[:, 1:H + 1, 1:W + 1, :]


def _border_mask(H, W):
    hh = jnp.arange(H + 2)
    ww = jnp.arange(W + 2)
    bh = (hh >= 1) & (hh <= H)
    bw = (ww >= 1) & (ww <= W)
    b = (bh[:, None] & bw[None, :]).astype(_F32)
    return b.reshape((H + 2) * (W + 2), 1)


def kernel(c2_feats, c3_feats, c4_feats, lat2_w, lat2_g, lat2_b, lat3_w,
           lat3_g, lat3_b, lat4_w, lat4_g, lat4_b, up43_w, up43_g, up43_b,
           up32_w, up32_g, up32_b, out_w, out_g, out_b, c2_coords, c3_coords,
           c4_coords):
    H4 = W4 = 24
    H3 = W3 = 47
    H2 = W2_ = 93
    P4 = (H4 + 2) * (W4 + 2)
    P3 = (H3 + 2) * (W3 + 2)
    P2 = (H2 + 2) * (W2_ + 2)
    W2p3 = W3 + 2
    W2p2 = W2_ + 2
    M3 = 2 * W2p3
    M2 = 2 * W2p2

    idx2 = _flat_padded_idx(c2_coords, H2, W2_)
    idx3 = _flat_padded_idx(c3_coords, H3, W3)
    idx4 = _flat_padded_idx(c4_coords, H4, W4)
    R2 = -(-P2 // 128) * 128
    R3 = -(-P3 // 128) * 128
    R4 = -(-P4 // 128) * 128
    y2, ix2 = _densify_prep(idx2, c2_feats, lat2_w)
    y3, ix3 = _densify_prep(idx3, c3_feats, lat3_w)
    y4, ix4 = _densify_prep(idx4, c4_feats, lat4_w)
    g2, c2c, g3, c3c, g4, c4c = _sc_scatter_all([
        (y2, ix2, P2, R2), (y3, ix3, P3, R3), (y4, ix4, P4, R4)])
    grid2, occ2 = _densify_split(g2, P2, R2), _densify_split(c2c, P2, R2)
    grid3, occ3 = _densify_split(g3, P3, R3), _densify_split(c3c, P3, R3)
    grid4, occ4 = _densify_split(g4, P4, R4), _densify_split(c4c, P4, R4)

    p2_lat, c2m = _bn_lateral(grid2, occ2, lat2_g, lat2_b, margin=M2)
    p3_lat, c3m = _bn_lateral(grid3, occ3, lat3_g, lat3_b)
    p4, c4m = _bn_lateral(grid4, occ4, lat4_g, lat4_b)

    border3 = _border_mask(H3, W3)
    border2 = _border_mask(H2, W2_)

    xa = _embed(_dilate2(_inner(p4, H4, W4)))
    ma = _embed(_dilate2(_inner(c4m, H4, W4)))
    raw = _conv(xa, None, up43_w, P3, W2p3, M3)
    p4_up_m = _mask_dilate(ma, None, border3, P3, W2p3, M3)
    p4_up = _bn_relu(raw, p4_up_m, None, up43_g, up43_b)
    p4_up = p4_up.reshape(2 * P3, 128)
    p4_up_m = p4_up_m.reshape(2 * P3, 1)

    xa = _embed(_dilate2(_inner(p3_lat, H3, W3)))
    xb = _embed(_dilate2(_inner(p4_up, H3, W3)))
    ma = _embed(_dilate2(_inner(c3m, H3, W3)))
    mb = _embed(_dilate2(_inner(p4_up_m, H3, W3)))
    raw = _conv(xa, xb, up32_w, P2, W2p2, M2)
    p3_up_m = _mask_dilate(ma, mb, border2, P2, W2p2, M2)
    p3_up = _bn_relu(raw, p3_up_m, None, up32_g, up32_b, margin=M2)

    raw = _conv(p2_lat, p3_up, out_w, P2, W2p2, M2)
    y = _bn_relu(raw, c2m.reshape(2, P2, 1), p3_up_m, out_g, out_b)

    out = y.reshape(2, H2 + 2, W2_ + 2, 128)[:, 1:H2 + 1, 1:W2_ + 1, :]
    return out[None]
